# baseline (speedup 1.0000x reference)
import functools

import jax
import jax.numpy as jnp
from jax import lax
from jax.experimental import pallas as pl
from jax.experimental.pallas import tpu as pltpu
from jax.experimental.pallas import tpu_sc as plsc

B, Wn, F, H, D, K, ALPHA = 4, 4096, 32, 2, 32, 16, 0.2
N = B * Wn
HD = H * D
NB = 2 * K + 1
NEG = -1e30
PADW = 128
NP = N + 2 * PADW

NT = 5120
NW = 32
NPW = (N - NT) // NW
L16 = 16
NG = NPW // L16
WIN = NPW + 2 * K

PBLK = 1664
TBLK = 512


def _proj_kernel(x_ref, ws_ref, wd_ref, fsT_ref, fdT_ref):
    xb = x_ref[...]
    dn = (((0,), (1,)), ((), ()))
    fsT_ref[...] = lax.dot_general(ws_ref[...], xb, dn,
                                   preferred_element_type=jnp.float32)
    fdT_ref[...] = lax.dot_general(wd_ref[...], xb, dn,
                                   preferred_element_type=jnp.float32)


NBP = 40


def _band_kernel(xp_ref, ws_ref, wd_ref, afc_ref, a2t_ref, sgt_ref, out_ref,
                 fsT_ref, qT_ref, u_ref, L0_ref, L1_ref):
    pid = pl.program_id(0)
    x_halo = xp_ref[pl.ds(pid * TBLK + PADW - K, TBLK + 2 * K), :]
    dn = (((0,), (1,)), ((), ()))
    fsT = lax.dot_general(ws_ref[...], x_halo, dn,
                          preferred_element_type=jnp.float32)
    fsT_ref[...] = fsT
    fdT = lax.dot_general(wd_ref[...], x_halo[K:K + TBLK], dn,
                          preferred_element_type=jnp.float32)
    afc = afc_ref[...]
    qT_ref[...] = fsT * afc
    rT = fdT * afc
    ssT = jnp.dot(a2t_ref[...], fsT, preferred_element_type=jnp.float32)
    sdT = jnp.dot(a2t_ref[...], fdT, preferred_element_type=jnp.float32)

    p0 = (pid % (Wn // TBLK)) * TBLK
    p = p0 + lax.broadcasted_iota(jnp.int32, (1, TBLK), 1)

    for o in range(NB):
        u_ref[:, o * TBLK:(o + 1) * TBLK] = jnp.abs(
            qT_ref[:, o:o + TBLK] + rT)
    t_all = jnp.dot(sgt_ref[...], u_ref[...],
                    preferred_element_type=jnp.float32)

    L0_ref[...] = jnp.full((NBP, TBLK), NEG, jnp.float32)
    L1_ref[...] = jnp.full((NBP, TBLK), NEG, jnp.float32)
    for o in range(NB):
        off = o - K
        lT = (0.6 * (ssT[:, o:o + TBLK] + sdT)
              + 0.4 * t_all[:, o * TBLK:(o + 1) * TBLK])
        valid = (p + off >= 0) & (p + off <= Wn - 1)
        L0_ref[o:o + 1, :] = jnp.where(valid, lT[0:1, :], NEG)
        L1_ref[o:o + 1, :] = jnp.where(valid, lT[1:2, :], NEG)

    L0, L1 = L0_ref[...], L1_ref[...]
    m0 = jnp.max(L0, axis=0, keepdims=True)
    m1 = jnp.max(L1, axis=0, keepdims=True)
    ex0 = jnp.exp(L0 - m0)
    ex1 = jnp.exp(L1 - m1)
    L0_ref[...] = ex0 * (0.5 / jnp.sum(ex0, axis=0, keepdims=True))
    L1_ref[...] = ex1 * (0.5 / jnp.sum(ex1, axis=0, keepdims=True))

    parts = [jnp.zeros((D, TBLK), dtype=jnp.float32) for _ in range(4)]
    for o in range(NB):
        fsT_sh = fsT_ref[:, o:o + TBLK]
        parts[o % 4] = (parts[o % 4] + fsT_sh[:D] * L0_ref[o:o + 1, :]
                        + fsT_sh[D:] * L1_ref[o:o + 1, :])
    acc = (parts[0] + parts[1]) + (parts[2] + parts[3])
    out_ref[...] = acc.T


_mesh = plsc.VectorSubcoreMesh(core_axis_name="c", subcore_axis_name="s")


@functools.partial(
    pl.kernel,
    out_type=jax.ShapeDtypeStruct((D, N - NT), jnp.float32),
    mesh=_mesh,
    compiler_params=pltpu.CompilerParams(use_tc_tiling_on_sc=False,
                                         needs_layout_passes=False),
    scratch_types=[
        pltpu.VMEM((HD, WIN), jnp.float32),
        pltpu.VMEM((HD, NPW), jnp.float32),
        pltpu.VMEM((HD, 128), jnp.float32),
        pltpu.VMEM((H, NB, L16), jnp.float32),
        pltpu.VMEM((H, NB, L16), jnp.float32),
        pltpu.VMEM((D, NPW), jnp.float32),
    ],
)
def _sc_attn(fsT_hbm, fdT_hbm, ab_hbm, out_hbm,
             fs_v, fd_v, a_v, lg_v, wg_v, ov_v):
    wid = lax.axis_index("s") * 2 + lax.axis_index("c")
    n0g = NT + wid * NPW
    pltpu.sync_copy(fsT_hbm.at[:, pl.ds(n0g + PADW - K, WIN)], fs_v)
    pltpu.sync_copy(fdT_hbm.at[:, pl.ds(n0g + PADW, NPW)], fd_v)
    pltpu.sync_copy(ab_hbm, a_v)

    lanes = jnp.arange(L16, dtype=jnp.int32)
    zero = jnp.zeros((L16,), jnp.float32)

    OCH = [(0, 8), (8, 8), (16, 8), (24, 8), (32, 1)]
    DCH = [(0, 16), (16, 16)]

    def group_body(g, carry):
        n0 = pl.multiple_of(g * L16, L16)
        pvec = (n0g + n0) % Wn + lanes
        col0 = n0 + lanes

        for h in range(H):
            mx = jnp.full((L16,), NEG, jnp.float32)
            for (co, cn) in OCH:
                def d_body(d, accs, h=h, co=co, cn=cn):
                    hd = h * D + d
                    fdv = fd_v[hd, pl.ds(n0, L16)]
                    av = a_v[hd, pl.ds(0, L16)]
                    row = jnp.full((L16,), hd, jnp.int32)
                    out = []
                    for i in range(cn):
                        fsv = plsc.load_gather(fs_v, [row, col0 + (co + i)])
                        t = fsv + fdv
                        t = jnp.maximum(t, ALPHA * t)
                        out.append(accs[i] + t * av)
                    return tuple(out)
                accs = lax.fori_loop(0, D, d_body, (zero,) * cn)
                for i in range(cn):
                    po = pvec + (co + i - K)
                    m = (po >= 0) & (po <= Wn - 1)
                    lm = jnp.where(m, accs[i], NEG)
                    lg_v[h, co + i, :] = lm
                    mx = jnp.maximum(mx, lm)

            den = zero
            for (co, cn) in OCH:
                exs = []
                for i in range(cn):
                    exs.append(jnp.exp(lg_v[h, co + i, :] - mx))
                for i in range(cn):
                    den = den + exs[i]
                    wg_v[h, co + i, :] = exs[i]
            inv = 0.5 / den
            for (co, cn) in OCH:
                vals = [wg_v[h, co + i, :] * inv for i in range(cn)]
                for i in range(cn):
                    wg_v[h, co + i, :] = vals[i]

        for (do, dn) in DCH:
            def o_body(o, accs, do=do, dn=dn):
                w0 = wg_v[0, o, :]
                w1 = wg_v[1, o, :]
                col = col0 + o
                out = []
                for i in range(dn):
                    f0 = plsc.load_gather(
                        fs_v, [jnp.full((L16,), do + i, jnp.int32), col])
                    f1 = plsc.load_gather(
                        fs_v, [jnp.full((L16,), D + do + i, jnp.int32), col])
                    out.append(accs[i] + (f0 * w0 + f1 * w1))
                return tuple(out)
            accs = lax.fori_loop(0, NB, o_body, (zero,) * dn)
            for i in range(dn):
                ov_v[do + i, pl.ds(n0, L16)] = accs[i]
        return carry

    lax.fori_loop(0, NG, group_body, 0)
    pltpu.sync_copy(ov_v, out_hbm.at[:, pl.ds(wid * NPW, NPW)])


@jax.jit
def _run(x, W_src, W_dst, attn_a, bias):
    nf = x.reshape(N, F)
    xp = jnp.pad(nf, ((PADW, PADW), (0, 0)))
    fsT, fdT = pl.pallas_call(
        _proj_kernel,
        grid=(NP // PBLK,),
        in_specs=[
            pl.BlockSpec((PBLK, F), lambda i: (i, 0)),
            pl.BlockSpec((F, HD), lambda i: (0, 0)),
            pl.BlockSpec((F, HD), lambda i: (0, 0)),
        ],
        out_specs=[
            pl.BlockSpec((HD, PBLK), lambda i: (0, i)),
            pl.BlockSpec((HD, PBLK), lambda i: (0, i)),
        ],
        out_shape=[
            jax.ShapeDtypeStruct((HD, NP), jnp.float32),
            jax.ShapeDtypeStruct((HD, NP), jnp.float32),
        ],
    )(xp, W_src, W_dst)
    a_b = jnp.broadcast_to(attn_a.reshape(HD, 1), (HD, 128))
    outT_sc = _sc_attn(fsT, fdT, a_b)

    af = attn_a.reshape(HD)
    afc = af.reshape(HD, 1)
    a2t = jnp.zeros((8, HD), jnp.float32)
    a2t = a2t.at[0, :D].set(af[:D]).at[1, D:].set(af[D:])
    sgt = jnp.zeros((8, HD), jnp.float32)
    sgt = sgt.at[0, :D].set(jnp.sign(af[:D])).at[1, D:].set(jnp.sign(af[D:]))
    out_tc = pl.pallas_call(
        _band_kernel,
        grid=(NT // TBLK,),
        in_specs=[
            pl.BlockSpec((NP, F), lambda i: (0, 0)),
            pl.BlockSpec((F, HD), lambda i: (0, 0)),
            pl.BlockSpec((F, HD), lambda i: (0, 0)),
            pl.BlockSpec((HD, 1), lambda i: (0, 0)),
            pl.BlockSpec((8, HD), lambda i: (0, 0)),
            pl.BlockSpec((8, HD), lambda i: (0, 0)),
        ],
        out_specs=pl.BlockSpec((TBLK, D), lambda i: (i, 0)),
        out_shape=jax.ShapeDtypeStruct((NT, D), jnp.float32),
        scratch_shapes=[
            pltpu.VMEM((HD, TBLK + 2 * K), jnp.float32),
            pltpu.VMEM((HD, TBLK + 2 * K), jnp.float32),
            pltpu.VMEM((HD, NB * TBLK), jnp.float32),
            pltpu.VMEM((NBP, TBLK), jnp.float32),
            pltpu.VMEM((NBP, TBLK), jnp.float32),
        ],
    )(xp, W_src, W_dst, afc, a2t, sgt)

    out = jnp.concatenate([out_tc, outT_sc.T], axis=0)
    out = out + bias.reshape(H, D).mean(axis=0)[None, :]
    return out.reshape(B, Wn, D)


def kernel(x, W_src, W_dst, attn_a, bias, src, dst):
    del src, dst
    return _run(x, W_src, W_dst, attn_a, bias)


# speedup vs baseline: 1.0481x; 1.0481x over previous
import functools

import jax
import jax.numpy as jnp
from jax import lax
from jax.experimental import pallas as pl
from jax.experimental.pallas import tpu as pltpu
from jax.experimental.pallas import tpu_sc as plsc

B, Wn, F, H, D, K, ALPHA = 4, 4096, 32, 2, 32, 16, 0.2
N = B * Wn
HD = H * D
NB = 2 * K + 1
NEG = -1e30
PADW = 128
NP = N + 2 * PADW

NT = 4608
NW = 32
NPW = (N - NT) // NW
L16 = 16
NG = NPW // L16
WIN = NPW + 2 * K

PBLK = 1664
TBLK = 512


def _proj_kernel(x_ref, ws_ref, wd_ref, fsT_ref, fdT_ref):
    xb = x_ref[...]
    dn = (((0,), (1,)), ((), ()))
    fsT_ref[...] = lax.dot_general(ws_ref[...], xb, dn,
                                   preferred_element_type=jnp.float32)
    fdT_ref[...] = lax.dot_general(wd_ref[...], xb, dn,
                                   preferred_element_type=jnp.float32)


NBP = 40


def _band_kernel(xp_ref, ws_ref, wd_ref, afc_ref, a2t_ref, sgt_ref, out_ref,
                 fsT_ref, qT_ref, u_ref, L0_ref, L1_ref):
    pid = pl.program_id(0)
    x_halo = xp_ref[pl.ds(pid * TBLK + PADW - K, TBLK + 2 * K), :]
    dn = (((0,), (1,)), ((), ()))
    fsT = lax.dot_general(ws_ref[...], x_halo, dn,
                          preferred_element_type=jnp.float32)
    fsT_ref[...] = fsT
    fdT = lax.dot_general(wd_ref[...], x_halo[K:K + TBLK], dn,
                          preferred_element_type=jnp.float32)
    afc = afc_ref[...]
    qT_ref[...] = fsT * afc
    rT = fdT * afc
    ssT = jnp.dot(a2t_ref[...], fsT, preferred_element_type=jnp.float32)
    sdT = jnp.dot(a2t_ref[...], fdT, preferred_element_type=jnp.float32)

    p0 = (pid % (Wn // TBLK)) * TBLK
    p = p0 + lax.broadcasted_iota(jnp.int32, (1, TBLK), 1)

    for o in range(NB):
        u_ref[:, o * TBLK:(o + 1) * TBLK] = jnp.abs(
            qT_ref[:, o:o + TBLK] + rT)
    t_all = jnp.dot(sgt_ref[...], u_ref[...],
                    preferred_element_type=jnp.float32)

    L0_ref[...] = jnp.full((NBP, TBLK), NEG, jnp.float32)
    L1_ref[...] = jnp.full((NBP, TBLK), NEG, jnp.float32)
    for o in range(NB):
        off = o - K
        lT = (0.6 * (ssT[:, o:o + TBLK] + sdT)
              + 0.4 * t_all[:, o * TBLK:(o + 1) * TBLK])
        valid = (p + off >= 0) & (p + off <= Wn - 1)
        L0_ref[o:o + 1, :] = jnp.where(valid, lT[0:1, :], NEG)
        L1_ref[o:o + 1, :] = jnp.where(valid, lT[1:2, :], NEG)

    L0, L1 = L0_ref[...], L1_ref[...]
    m0 = jnp.max(L0, axis=0, keepdims=True)
    m1 = jnp.max(L1, axis=0, keepdims=True)
    ex0 = jnp.exp(L0 - m0)
    ex1 = jnp.exp(L1 - m1)
    L0_ref[...] = ex0 * (0.5 / jnp.sum(ex0, axis=0, keepdims=True))
    L1_ref[...] = ex1 * (0.5 / jnp.sum(ex1, axis=0, keepdims=True))

    parts = [jnp.zeros((D, TBLK), dtype=jnp.float32) for _ in range(4)]
    for o in range(NB):
        fsT_sh = fsT_ref[:, o:o + TBLK]
        parts[o % 4] = (parts[o % 4] + fsT_sh[:D] * L0_ref[o:o + 1, :]
                        + fsT_sh[D:] * L1_ref[o:o + 1, :])
    acc = (parts[0] + parts[1]) + (parts[2] + parts[3])
    out_ref[...] = acc.T


_mesh = plsc.VectorSubcoreMesh(core_axis_name="c", subcore_axis_name="s")


@functools.partial(
    pl.kernel,
    out_type=jax.ShapeDtypeStruct((D, N - NT), jnp.float32),
    mesh=_mesh,
    compiler_params=pltpu.CompilerParams(use_tc_tiling_on_sc=False,
                                         needs_layout_passes=False),
    scratch_types=[
        pltpu.VMEM((HD, WIN), jnp.float32),
        pltpu.VMEM((HD, NPW), jnp.float32),
        pltpu.VMEM((HD, 128), jnp.float32),
        pltpu.VMEM((H, NB, L16), jnp.float32),
        pltpu.VMEM((H, NB, L16), jnp.float32),
        pltpu.VMEM((D, NPW), jnp.float32),
    ],
)
def _sc_attn(fsT_hbm, fdT_hbm, ab_hbm, out_hbm,
             fs_v, fd_v, a_v, lg_v, wg_v, ov_v):
    wid = lax.axis_index("s") * 2 + lax.axis_index("c")
    n0g = NT + wid * NPW
    pltpu.sync_copy(fsT_hbm.at[:, pl.ds(n0g + PADW - K, WIN)], fs_v)
    pltpu.sync_copy(fdT_hbm.at[:, pl.ds(n0g + PADW, NPW)], fd_v)
    pltpu.sync_copy(ab_hbm, a_v)

    lanes = jnp.arange(L16, dtype=jnp.int32)
    zero = jnp.zeros((L16,), jnp.float32)

    OCH = [(0, 8), (8, 8), (16, 8), (24, 8), (32, 1)]
    DCH = [(0, 16), (16, 16)]

    def group_body(g, carry):
        n0 = pl.multiple_of(g * L16, L16)
        pvec = (n0g + n0) % Wn + lanes
        col0 = n0 + lanes

        for h in range(H):
            mx = jnp.full((L16,), NEG, jnp.float32)
            for (co, cn) in OCH:
                def d_body(d, accs, h=h, co=co, cn=cn):
                    hd = h * D + d
                    fdv = fd_v[hd, pl.ds(n0, L16)]
                    av = a_v[hd, pl.ds(0, L16)]
                    row = jnp.full((L16,), hd, jnp.int32)
                    out = []
                    for i in range(cn):
                        fsv = plsc.load_gather(fs_v, [row, col0 + (co + i)])
                        t = fsv + fdv
                        t = jnp.maximum(t, ALPHA * t)
                        out.append(accs[i] + t * av)
                    return tuple(out)
                accs = lax.fori_loop(0, D, d_body, (zero,) * cn)
                for i in range(cn):
                    po = pvec + (co + i - K)
                    m = (po >= 0) & (po <= Wn - 1)
                    lm = jnp.where(m, accs[i], NEG)
                    lg_v[h, co + i, :] = lm
                    mx = jnp.maximum(mx, lm)

            den = zero
            for (co, cn) in OCH:
                exs = []
                for i in range(cn):
                    exs.append(jnp.exp(lg_v[h, co + i, :] - mx))
                for i in range(cn):
                    den = den + exs[i]
                    wg_v[h, co + i, :] = exs[i]
            inv = 0.5 / den
            for (co, cn) in OCH:
                vals = [wg_v[h, co + i, :] * inv for i in range(cn)]
                for i in range(cn):
                    wg_v[h, co + i, :] = vals[i]

        for (do, dn) in DCH:
            def o_body(o, accs, do=do, dn=dn):
                w0 = wg_v[0, o, :]
                w1 = wg_v[1, o, :]
                col = col0 + o
                out = []
                for i in range(dn):
                    f0 = plsc.load_gather(
                        fs_v, [jnp.full((L16,), do + i, jnp.int32), col])
                    f1 = plsc.load_gather(
                        fs_v, [jnp.full((L16,), D + do + i, jnp.int32), col])
                    out.append(accs[i] + (f0 * w0 + f1 * w1))
                return tuple(out)
            accs = lax.fori_loop(0, NB, o_body, (zero,) * dn)
            for i in range(dn):
                ov_v[do + i, pl.ds(n0, L16)] = accs[i]
        return carry

    lax.fori_loop(0, NG, group_body, 0)
    pltpu.sync_copy(ov_v, out_hbm.at[:, pl.ds(wid * NPW, NPW)])


@jax.jit
def _run(x, W_src, W_dst, attn_a, bias):
    nf = x.reshape(N, F)
    xp = jnp.pad(nf, ((PADW, PADW), (0, 0)))
    fsT, fdT = pl.pallas_call(
        _proj_kernel,
        grid=(NP // PBLK,),
        in_specs=[
            pl.BlockSpec((PBLK, F), lambda i: (i, 0)),
            pl.BlockSpec((F, HD), lambda i: (0, 0)),
            pl.BlockSpec((F, HD), lambda i: (0, 0)),
        ],
        out_specs=[
            pl.BlockSpec((HD, PBLK), lambda i: (0, i)),
            pl.BlockSpec((HD, PBLK), lambda i: (0, i)),
        ],
        out_shape=[
            jax.ShapeDtypeStruct((HD, NP), jnp.float32),
            jax.ShapeDtypeStruct((HD, NP), jnp.float32),
        ],
    )(xp, W_src, W_dst)
    a_b = jnp.broadcast_to(attn_a.reshape(HD, 1), (HD, 128))
    outT_sc = _sc_attn(fsT, fdT, a_b)

    af = attn_a.reshape(HD)
    afc = af.reshape(HD, 1)
    a2t = jnp.zeros((8, HD), jnp.float32)
    a2t = a2t.at[0, :D].set(af[:D]).at[1, D:].set(af[D:])
    sgt = jnp.zeros((8, HD), jnp.float32)
    sgt = sgt.at[0, :D].set(jnp.sign(af[:D])).at[1, D:].set(jnp.sign(af[D:]))
    out_tc = pl.pallas_call(
        _band_kernel,
        grid=(NT // TBLK,),
        in_specs=[
            pl.BlockSpec((NP, F), lambda i: (0, 0)),
            pl.BlockSpec((F, HD), lambda i: (0, 0)),
            pl.BlockSpec((F, HD), lambda i: (0, 0)),
            pl.BlockSpec((HD, 1), lambda i: (0, 0)),
            pl.BlockSpec((8, HD), lambda i: (0, 0)),
            pl.BlockSpec((8, HD), lambda i: (0, 0)),
        ],
        out_specs=pl.BlockSpec((TBLK, D), lambda i: (i, 0)),
        out_shape=jax.ShapeDtypeStruct((NT, D), jnp.float32),
        scratch_shapes=[
            pltpu.VMEM((HD, TBLK + 2 * K), jnp.float32),
            pltpu.VMEM((HD, TBLK + 2 * K), jnp.float32),
            pltpu.VMEM((HD, NB * TBLK), jnp.float32),
            pltpu.VMEM((NBP, TBLK), jnp.float32),
            pltpu.VMEM((NBP, TBLK), jnp.float32),
        ],
    )(xp, W_src, W_dst, afc, a2t, sgt)

    out = jnp.concatenate([out_tc, outT_sc.T], axis=0)
    out = out + bias.reshape(H, D).mean(axis=0)[None, :]
    return out.reshape(B, Wn, D)


def kernel(x, W_src, W_dst, attn_a, bias, src, dst):
    del src, dst
    return _run(x, W_src, W_dst, attn_a, bias)
